# baseline (speedup 1.0000x reference)
import functools
import math

import jax
import jax.numpy as jnp
from jax import lax
from jax.experimental import pallas as pl
from jax.experimental.pallas import tpu as pltpu
from jax.experimental.pallas import tpu_sc as plsc

N_NODES = 10000
N_PAD = 10240
N_EDGES = 320000
D = 128
H = 64
D_TIME = 16
NC = 2
NS = 16
NW = NC * NS
INV_SCALE = 1.0 / math.sqrt(2 * D + D_TIME)


def _tc_attn_body(hs_ref, hd_ref, ws_ref, wd_ref,
                  as_ref, ad_ref, es_ref, ed_ref):
    ws = jnp.dot(ws_ref[...], as_ref[...], preferred_element_type=jnp.float32)
    es_ref[...] = jnp.dot(hs_ref[...], ws, preferred_element_type=jnp.float32)
    wd = jnp.dot(wd_ref[...], ad_ref[...], preferred_element_type=jnp.float32)
    ed_ref[...] = jnp.dot(hd_ref[...], wd, preferred_element_type=jnp.float32)


def _tc_attn(h_src, h_dst, W_src, W_dst, a_s, a_d):
    return pl.pallas_call(
        _tc_attn_body,
        out_shape=[
            jax.ShapeDtypeStruct((N_NODES, 1), jnp.float32),
            jax.ShapeDtypeStruct((N_NODES, 1), jnp.float32),
        ],
    )(h_src, h_dst, W_src, W_dst, a_s, a_d)


_ET_GRID = 50
_ET_BLK = N_EDGES // _ET_GRID


def _tc_et_body(at_ref, phi_ref, et_ref):
    et_ref[...] = lax.dot_general(at_ref[...], phi_ref[...],
                                  (((1,), (1,)), ((), ())),
                                  preferred_element_type=jnp.float32)


def _tc_et(phi_dt, a_t2):
    return pl.pallas_call(
        _tc_et_body,
        grid=(_ET_GRID,),
        in_specs=[
            pl.BlockSpec((1, D_TIME), lambda i: (0, 0)),
            pl.BlockSpec((_ET_BLK, D_TIME), lambda i: (i, 0)),
        ],
        out_specs=pl.BlockSpec((1, _ET_BLK), lambda i: (0, i)),
        out_shape=jax.ShapeDtypeStruct((1, N_EDGES), jnp.float32),
    )(a_t2, phi_dt)


def _tc_zs(h_src, W_src):
    return pl.pallas_call(
        lambda h_ref, w_ref, o_ref: o_ref.__setitem__(
            ..., jnp.dot(h_ref[...], w_ref[...],
                         preferred_element_type=jnp.float32)),
        out_shape=jax.ShapeDtypeStruct((N_NODES, D), jnp.float32),
    )(h_src, W_src)


_EPT = N_EDGES // NW
_C1 = 2000
_NCH1 = _EPT // _C1
_SLAB = N_PAD // NS


def _zero_1d(ref, nvec):
    z = jnp.zeros((16,), jnp.float32)

    def body(i, _):
        ref[pl.ds(i * 16, 16)] = z
        return 0

    lax.fori_loop(0, nvec, body, 0, unroll=4)


_sc_mesh = plsc.VectorSubcoreMesh(core_axis_name="c", subcore_axis_name="s")
_sc_params = pltpu.CompilerParams(needs_layout_passes=False)


@functools.partial(
    pl.kernel,
    out_type=[
        jax.ShapeDtypeStruct((N_EDGES,), jnp.float32),
        jax.ShapeDtypeStruct((NC * N_PAD,), jnp.float32),
    ],
    mesh=_sc_mesh,
    compiler_params=_sc_params,
    scratch_types=[
        pltpu.VMEM((N_PAD,), jnp.float32),
        pltpu.VMEM((N_PAD,), jnp.float32),
        pltpu.VMEM((N_PAD,), jnp.float32),
        pltpu.VMEM((_C1,), jnp.int32),
        pltpu.VMEM((_C1,), jnp.int32),
        pltpu.VMEM((_C1,), jnp.float32),
        pltpu.VMEM((_C1,), jnp.float32),
        pltpu.VMEM((_SLAB,), jnp.float32),
        pltpu.VMEM((_SLAB,), jnp.float32),
        pltpu.VMEM_SHARED((NS * N_PAD,), jnp.float32),
    ],
)
def _sc_scalar(ei_hbm, et_hbm, es_hbm, ed_hbm, expe_hbm, psums_hbm,
               es_v, ed_v, psum_v, srcv, dstv, etv, pbuf, red_v, tmp_v,
               shared):
    c = lax.axis_index("c")
    s = lax.axis_index("s")
    wid = s * NC + c
    base = wid * _EPT

    pltpu.sync_copy(es_hbm, es_v.at[pl.ds(0, N_NODES)])
    pltpu.sync_copy(ed_hbm, ed_v.at[pl.ds(0, N_NODES)])
    _zero_1d(psum_v, N_PAD // 16)

    for ch in range(_NCH1):
        off = base + ch * _C1
        pltpu.sync_copy(ei_hbm.at[pl.ds(off, _C1)], srcv)
        pltpu.sync_copy(ei_hbm.at[pl.ds(N_EDGES + off, _C1)], dstv)
        pltpu.sync_copy(et_hbm.at[pl.ds(off, _C1)], etv)

        def vbody(v, _):
            sl = pl.ds(v * 16, 16)
            sv = srcv[sl]
            dv = dstv[sl]
            e = plsc.load_gather(es_v, [sv]) + plsc.load_gather(ed_v, [dv]) + etv[sl]
            e = jnp.where(e >= 0.0, e, 0.2 * e) * INV_SCALE
            p = jnp.exp(e)
            pbuf[sl] = p
            plsc.addupdate_scatter(psum_v, [dv], p)
            return 0

        lax.fori_loop(0, _C1 // 16, vbody, 0, unroll=2)
        pltpu.sync_copy(pbuf, expe_hbm.at[pl.ds(off, _C1)])

    pltpu.sync_copy(psum_v, shared.at[pl.ds(s * N_PAD, N_PAD)])
    plsc.subcore_barrier()
    sbase = s * _SLAB
    _zero_1d(red_v, _SLAB // 16)
    for k in range(NS):
        pltpu.sync_copy(shared.at[pl.ds(k * N_PAD + sbase, _SLAB)], tmp_v)

        def abody(i, _):
            sl = pl.ds(i * 16, 16)
            red_v[sl] = red_v[sl] + tmp_v[sl]
            return 0

        lax.fori_loop(0, _SLAB // 16, abody, 0, unroll=4)
    pltpu.sync_copy(red_v, psums_hbm.at[pl.ds(c * N_PAD + sbase, _SLAB)])


_C2 = 128
_EPC = N_EDGES // NC
_NCH2 = _EPC // _C2


@functools.partial(
    pl.kernel,
    out_type=jax.ShapeDtypeStruct((NC, N_NODES, D), jnp.float32),
    mesh=_sc_mesh,
    compiler_params=_sc_params,
    scratch_types=[
        pltpu.VMEM((_C2,), jnp.int32),
        pltpu.VMEM((_C2,), jnp.int32),
        pltpu.VMEM((_C2,), jnp.float32),
        pltpu.VMEM((_C2, D), jnp.float32),
        pltpu.VMEM((_C2,), jnp.int32),
        pltpu.VMEM((_C2,), jnp.int32),
        pltpu.VMEM((_C2,), jnp.float32),
        pltpu.VMEM((_C2, D), jnp.float32),
        pltpu.VMEM((_C2,), jnp.int32),
        pltpu.VMEM((_C2,), jnp.int32),
        pltpu.VMEM_SHARED((N_PAD, D), jnp.float32),
        pltpu.SemaphoreType.DMA,
        pltpu.SemaphoreType.DMA,
        pltpu.SemaphoreType.DMA,
        pltpu.SemaphoreType.DMA,
        pltpu.SemaphoreType.DMA,
        pltpu.SemaphoreType.DMA,
    ],
)
def _sc_vector(ei_hbm, expe_hbm, zs_hbm, part_hbm,
               srcA, dstA, pvA, rowsA,
               srcB, dstB, pvB, rowsB, dstSA, dstSB, acc,
               semA, semB, semSA, semSB, semIA, semIB):
    c = lax.axis_index("c")
    s = lax.axis_index("s")
    ebase = c * _EPC

    def zrow(i, _):
        for k in range(D // 16):
            rowsA[i, pl.ds(k * 16, 16)] = jnp.zeros((16,), jnp.float32)
        return 0

    lax.fori_loop(0, _C2, zrow, 0, unroll=4)
    for k in range(_SLAB // _C2):
        pltpu.sync_copy(rowsA, acc.at[pl.ds(s * _SLAB + k * _C2, _C2)])

    plsc.subcore_barrier()

    n_it = jnp.where(s < _NCH2 % NS, _NCH2 // NS + 1, _NCH2 // NS)
    n_pair = (n_it + 1) // 2

    def fetch_idx(it, src_b, dst_b, pv_b):
        off = ebase + (s + it * NS) * _C2
        pltpu.sync_copy(ei_hbm.at[pl.ds(off, _C2)], src_b)
        pltpu.sync_copy(ei_hbm.at[pl.ds(N_EDGES + off, _C2)], dst_b)
        pltpu.sync_copy(expe_hbm.at[pl.ds(off, _C2)], pv_b)

    def fetch_idx_async(it, src_b, dst_b, pv_b, sem_i):
        off = ebase + (s + it * NS) * _C2
        pltpu.async_copy(ei_hbm.at[pl.ds(off, _C2)], src_b, sem_i)
        pltpu.async_copy(ei_hbm.at[pl.ds(N_EDGES + off, _C2)], dst_b, sem_i)
        pltpu.async_copy(expe_hbm.at[pl.ds(off, _C2)], pv_b, sem_i)

    def wait_idx(it, src_b, dst_b, pv_b, sem_i):
        off = ebase + (s + it * NS) * _C2
        pltpu.make_async_copy(ei_hbm.at[pl.ds(off, _C2)], src_b, sem_i).wait()
        pltpu.make_async_copy(ei_hbm.at[pl.ds(N_EDGES + off, _C2)], dst_b, sem_i).wait()
        pltpu.make_async_copy(expe_hbm.at[pl.ds(off, _C2)], pv_b, sem_i).wait()

    def process(dst_b, pv_b, rows_b, dstS_b, sem_s):
        for k in range(_C2 // 16):
            sl = pl.ds(k * 16, 16)
            dstS_b[sl] = dst_b[sl]

        def scale(j, _):
            aj = plsc.load_gather(pv_b, [jnp.full((16,), 0, jnp.int32) + j])
            for k in range(D // 16):
                sl = pl.ds(k * 16, 16)
                rows_b[j, sl] = rows_b[j, sl] * aj
            return 0

        lax.fori_loop(0, _C2, scale, 0, unroll=4)
        pltpu.async_copy(rows_b, acc.at[dstS_b], sem_s, add=True)

    def drain_scatter(rows_b, sem_s):
        pltpu.make_async_copy(zs_hbm.at[pl.ds(0, _C2)], rows_b, sem_s).wait()

    fetch_idx(0, srcA, dstA, pvA)
    pltpu.async_copy(zs_hbm.at[srcA], rowsA, semA)
    fetch_idx(1, srcB, dstB, pvB)
    pltpu.async_copy(zs_hbm.at[srcB], rowsB, semB)

    def pair(t, _):
        it0 = 2 * t
        it1 = it0 + 1

        pltpu.make_async_copy(zs_hbm.at[srcA], rowsA, semA).wait()
        process(dstA, pvA, rowsA, dstSA, semSA)

        @pl.when(it0 + 2 < n_it)
        def _():
            fetch_idx_async(it0 + 2, srcA, dstA, pvA, semIA)

        @pl.when(it1 < n_it)
        def _():
            pltpu.make_async_copy(zs_hbm.at[srcB], rowsB, semB).wait()
            process(dstB, pvB, rowsB, dstSB, semSB)

        @pl.when(it1 + 2 < n_it)
        def _():
            fetch_idx_async(it1 + 2, srcB, dstB, pvB, semIB)

        @pl.when(it0 + 2 < n_it)
        def _():
            drain_scatter(rowsA, semSA)
            wait_idx(it0 + 2, srcA, dstA, pvA, semIA)
            pltpu.async_copy(zs_hbm.at[srcA], rowsA, semA)

        @pl.when(it1 + 2 < n_it)
        def _():
            drain_scatter(rowsB, semSB)
            wait_idx(it1 + 2, srcB, dstB, pvB, semIB)
            pltpu.async_copy(zs_hbm.at[srcB], rowsB, semB)

        return 0

    lax.fori_loop(0, n_pair, pair, 0)

    drain_scatter(rowsA, semSA)
    drain_scatter(rowsB, semSB)
    plsc.subcore_barrier()
    base = s * _SLAB
    tail = N_NODES - (NS - 1) * _SLAB

    @pl.when(s < NS - 1)
    def _():
        pltpu.sync_copy(acc.at[pl.ds(base, _SLAB)],
                        part_hbm.at[c, pl.ds(base, _SLAB)])

    @pl.when(s == NS - 1)
    def _():
        pltpu.sync_copy(acc.at[pl.ds((NS - 1) * _SLAB, tail)],
                        part_hbm.at[c, pl.ds((NS - 1) * _SLAB, tail)])


def _tc_norm_body(p_ref, s0_ref, s1_ref, o_ref):
    denom = s0_ref[...] + s1_ref[...] + 1e-12
    o_ref[...] = (p_ref[0] + p_ref[1]) * (1.0 / denom)


def _tc_norm(parts, s0, s1):
    return pl.pallas_call(
        _tc_norm_body,
        out_shape=jax.ShapeDtypeStruct((N_NODES, D), jnp.float32),
    )(parts, s0, s1)


def kernel(h_src, h_dst, edge_index, phi_dt, W_src, W_dst, a):
    a_s = a[:D].reshape(D, 1)
    a_d = a[D:2 * D].reshape(D, 1)
    a_t = a[2 * D:]

    es, ed = _tc_attn(h_src, h_dst, W_src, W_dst, a_s, a_d)
    et = _tc_et(phi_dt, a_t.reshape(1, D_TIME))
    zs = _tc_zs(h_src, W_src)
    et = et.reshape(N_EDGES)
    es = es.reshape(N_NODES)
    ed = ed.reshape(N_NODES)

    ei_flat = edge_index.reshape(2 * N_EDGES)
    exp_e, psums = _sc_scalar(ei_flat, et, es, ed)
    parts = _sc_vector(ei_flat, exp_e, zs)
    s0 = psums[:N_NODES].reshape(N_NODES, 1)
    s1 = psums[N_PAD:N_PAD + N_NODES].reshape(N_NODES, 1)
    return _tc_norm(parts, s0, s1)


# speedup vs baseline: 1.0136x; 1.0136x over previous
import functools
import math

import jax
import jax.numpy as jnp
from jax import lax
from jax.experimental import pallas as pl
from jax.experimental.pallas import tpu as pltpu
from jax.experimental.pallas import tpu_sc as plsc

N_NODES = 10000
N_PAD = 10240
N_EDGES = 320000
D = 128
H = 64
D_TIME = 16
NC = 2
NS = 16
NW = NC * NS
INV_SCALE = 1.0 / math.sqrt(2 * D + D_TIME)


_PHI_ROWS = N_EDGES // D
_PHI_COLS = D * D_TIME


def _tc_attn_body(phi_ref, at_blk_ref, hs_ref, hd_ref, ws_ref, wd_ref,
                  as_ref, ad_ref, et_ref, es_ref, ed_ref):
    et_ref[...] = jnp.dot(phi_ref[...], at_blk_ref[...],
                          preferred_element_type=jnp.float32)
    ws = jnp.dot(ws_ref[...], as_ref[...], preferred_element_type=jnp.float32)
    es_ref[...] = jnp.dot(hs_ref[...], ws, preferred_element_type=jnp.float32)
    wd = jnp.dot(wd_ref[...], ad_ref[...], preferred_element_type=jnp.float32)
    ed_ref[...] = jnp.dot(hd_ref[...], wd, preferred_element_type=jnp.float32)


def _tc_attn(h_src, h_dst, W_src, W_dst, a_s, a_d, at_blk, phi2):
    return pl.pallas_call(
        _tc_attn_body,
        out_shape=[
            jax.ShapeDtypeStruct((_PHI_ROWS, D), jnp.float32),
            jax.ShapeDtypeStruct((N_NODES, 1), jnp.float32),
            jax.ShapeDtypeStruct((N_NODES, 1), jnp.float32),
        ],
    )(phi2, at_blk, h_src, h_dst, W_src, W_dst, a_s, a_d)


def _tc_zs(h_src, W_src):
    return pl.pallas_call(
        lambda h_ref, w_ref, o_ref: o_ref.__setitem__(
            ..., jnp.dot(h_ref[...], w_ref[...],
                         preferred_element_type=jnp.float32)),
        out_shape=jax.ShapeDtypeStruct((N_NODES, D), jnp.float32),
    )(h_src, W_src)


_EPT = N_EDGES // NW
_C1 = 2000
_NCH1 = _EPT // _C1
_SLAB = N_PAD // NS


def _zero_1d(ref, nvec):
    z = jnp.zeros((16,), jnp.float32)

    def body(i, _):
        ref[pl.ds(i * 16, 16)] = z
        return 0

    lax.fori_loop(0, nvec, body, 0, unroll=4)


_sc_mesh = plsc.VectorSubcoreMesh(core_axis_name="c", subcore_axis_name="s")
_sc_params = pltpu.CompilerParams(needs_layout_passes=False)


@functools.partial(
    pl.kernel,
    out_type=[
        jax.ShapeDtypeStruct((N_EDGES,), jnp.float32),
        jax.ShapeDtypeStruct((NC * N_PAD,), jnp.float32),
    ],
    mesh=_sc_mesh,
    compiler_params=_sc_params,
    scratch_types=[
        pltpu.VMEM((N_PAD,), jnp.float32),
        pltpu.VMEM((N_PAD,), jnp.float32),
        pltpu.VMEM((N_PAD,), jnp.float32),
        pltpu.VMEM((_C1,), jnp.int32),
        pltpu.VMEM((_C1,), jnp.int32),
        pltpu.VMEM((_C1,), jnp.float32),
        pltpu.VMEM((_C1,), jnp.float32),
        pltpu.VMEM((_SLAB,), jnp.float32),
        pltpu.VMEM((_SLAB,), jnp.float32),
        pltpu.VMEM_SHARED((NS * N_PAD,), jnp.float32),
    ],
)
def _sc_scalar(ei_hbm, et_hbm, es_hbm, ed_hbm, expe_hbm, psums_hbm,
               es_v, ed_v, psum_v, srcv, dstv, etv, pbuf, red_v, tmp_v,
               shared):
    c = lax.axis_index("c")
    s = lax.axis_index("s")
    wid = s * NC + c
    base = wid * _EPT

    pltpu.sync_copy(es_hbm, es_v.at[pl.ds(0, N_NODES)])
    pltpu.sync_copy(ed_hbm, ed_v.at[pl.ds(0, N_NODES)])
    _zero_1d(psum_v, N_PAD // 16)

    for ch in range(_NCH1):
        off = base + ch * _C1
        pltpu.sync_copy(ei_hbm.at[pl.ds(off, _C1)], srcv)
        pltpu.sync_copy(ei_hbm.at[pl.ds(N_EDGES + off, _C1)], dstv)
        pltpu.sync_copy(et_hbm.at[pl.ds(off, _C1)], etv)

        def vbody(v, _):
            sl = pl.ds(v * 16, 16)
            sv = srcv[sl]
            dv = dstv[sl]
            e = plsc.load_gather(es_v, [sv]) + plsc.load_gather(ed_v, [dv]) + etv[sl]
            e = jnp.where(e >= 0.0, e, 0.2 * e) * INV_SCALE
            p = jnp.exp(e)
            pbuf[sl] = p
            plsc.addupdate_scatter(psum_v, [dv], p)
            return 0

        lax.fori_loop(0, _C1 // 16, vbody, 0, unroll=2)
        pltpu.sync_copy(pbuf, expe_hbm.at[pl.ds(off, _C1)])

    pltpu.sync_copy(psum_v, shared.at[pl.ds(s * N_PAD, N_PAD)])
    plsc.subcore_barrier()
    sbase = s * _SLAB
    _zero_1d(red_v, _SLAB // 16)
    for k in range(NS):
        pltpu.sync_copy(shared.at[pl.ds(k * N_PAD + sbase, _SLAB)], tmp_v)

        def abody(i, _):
            sl = pl.ds(i * 16, 16)
            red_v[sl] = red_v[sl] + tmp_v[sl]
            return 0

        lax.fori_loop(0, _SLAB // 16, abody, 0, unroll=4)
    pltpu.sync_copy(red_v, psums_hbm.at[pl.ds(c * N_PAD + sbase, _SLAB)])


_C2 = 128
_EPC = N_EDGES // NC
_NCH2 = _EPC // _C2


@functools.partial(
    pl.kernel,
    out_type=jax.ShapeDtypeStruct((NC, N_NODES, D), jnp.float32),
    mesh=_sc_mesh,
    compiler_params=_sc_params,
    scratch_types=[
        pltpu.VMEM((_C2,), jnp.int32),
        pltpu.VMEM((_C2,), jnp.int32),
        pltpu.VMEM((_C2,), jnp.float32),
        pltpu.VMEM((_C2, D), jnp.float32),
        pltpu.VMEM((_C2,), jnp.int32),
        pltpu.VMEM((_C2,), jnp.int32),
        pltpu.VMEM((_C2,), jnp.float32),
        pltpu.VMEM((_C2, D), jnp.float32),
        pltpu.VMEM((_C2,), jnp.int32),
        pltpu.VMEM((_C2,), jnp.int32),
        pltpu.VMEM_SHARED((N_PAD, D), jnp.float32),
        pltpu.SemaphoreType.DMA,
        pltpu.SemaphoreType.DMA,
        pltpu.SemaphoreType.DMA,
        pltpu.SemaphoreType.DMA,
        pltpu.SemaphoreType.DMA,
        pltpu.SemaphoreType.DMA,
    ],
)
def _sc_vector(ei_hbm, expe_hbm, zs_hbm, part_hbm,
               srcA, dstA, pvA, rowsA,
               srcB, dstB, pvB, rowsB, dstSA, dstSB, acc,
               semA, semB, semSA, semSB, semIA, semIB):
    c = lax.axis_index("c")
    s = lax.axis_index("s")
    ebase = c * _EPC

    def zrow(i, _):
        for k in range(D // 16):
            rowsA[i, pl.ds(k * 16, 16)] = jnp.zeros((16,), jnp.float32)
        return 0

    lax.fori_loop(0, _C2, zrow, 0, unroll=4)
    for k in range(_SLAB // _C2):
        pltpu.sync_copy(rowsA, acc.at[pl.ds(s * _SLAB + k * _C2, _C2)])

    plsc.subcore_barrier()

    n_it = jnp.where(s < _NCH2 % NS, _NCH2 // NS + 1, _NCH2 // NS)
    n_pair = (n_it + 1) // 2

    def fetch_idx(it, src_b, dst_b, pv_b):
        off = ebase + (s + it * NS) * _C2
        pltpu.sync_copy(ei_hbm.at[pl.ds(off, _C2)], src_b)
        pltpu.sync_copy(ei_hbm.at[pl.ds(N_EDGES + off, _C2)], dst_b)
        pltpu.sync_copy(expe_hbm.at[pl.ds(off, _C2)], pv_b)

    def fetch_idx_async(it, src_b, dst_b, pv_b, sem_i):
        off = ebase + (s + it * NS) * _C2
        pltpu.async_copy(ei_hbm.at[pl.ds(off, _C2)], src_b, sem_i)
        pltpu.async_copy(ei_hbm.at[pl.ds(N_EDGES + off, _C2)], dst_b, sem_i)
        pltpu.async_copy(expe_hbm.at[pl.ds(off, _C2)], pv_b, sem_i)

    def wait_idx(it, src_b, dst_b, pv_b, sem_i):
        off = ebase + (s + it * NS) * _C2
        pltpu.make_async_copy(ei_hbm.at[pl.ds(off, _C2)], src_b, sem_i).wait()
        pltpu.make_async_copy(ei_hbm.at[pl.ds(N_EDGES + off, _C2)], dst_b, sem_i).wait()
        pltpu.make_async_copy(expe_hbm.at[pl.ds(off, _C2)], pv_b, sem_i).wait()

    def process(dst_b, pv_b, rows_b, dstS_b, sem_s):
        for k in range(_C2 // 16):
            sl = pl.ds(k * 16, 16)
            dstS_b[sl] = dst_b[sl]

        def scale(j, _):
            aj = plsc.load_gather(pv_b, [jnp.full((16,), 0, jnp.int32) + j])
            for k in range(D // 16):
                sl = pl.ds(k * 16, 16)
                rows_b[j, sl] = rows_b[j, sl] * aj
            return 0

        lax.fori_loop(0, _C2, scale, 0, unroll=4)
        pltpu.async_copy(rows_b, acc.at[dstS_b], sem_s, add=True)

    def drain_scatter(rows_b, sem_s):
        pltpu.make_async_copy(zs_hbm.at[pl.ds(0, _C2)], rows_b, sem_s).wait()

    fetch_idx(0, srcA, dstA, pvA)
    pltpu.async_copy(zs_hbm.at[srcA], rowsA, semA)
    fetch_idx(1, srcB, dstB, pvB)
    pltpu.async_copy(zs_hbm.at[srcB], rowsB, semB)

    def pair(t, _):
        it0 = 2 * t
        it1 = it0 + 1

        pltpu.make_async_copy(zs_hbm.at[srcA], rowsA, semA).wait()
        process(dstA, pvA, rowsA, dstSA, semSA)

        @pl.when(it0 + 2 < n_it)
        def _():
            fetch_idx_async(it0 + 2, srcA, dstA, pvA, semIA)

        @pl.when(it1 < n_it)
        def _():
            pltpu.make_async_copy(zs_hbm.at[srcB], rowsB, semB).wait()
            process(dstB, pvB, rowsB, dstSB, semSB)

        @pl.when(it1 + 2 < n_it)
        def _():
            fetch_idx_async(it1 + 2, srcB, dstB, pvB, semIB)

        @pl.when(it0 + 2 < n_it)
        def _():
            drain_scatter(rowsA, semSA)
            wait_idx(it0 + 2, srcA, dstA, pvA, semIA)
            pltpu.async_copy(zs_hbm.at[srcA], rowsA, semA)

        @pl.when(it1 + 2 < n_it)
        def _():
            drain_scatter(rowsB, semSB)
            wait_idx(it1 + 2, srcB, dstB, pvB, semIB)
            pltpu.async_copy(zs_hbm.at[srcB], rowsB, semB)

        return 0

    lax.fori_loop(0, n_pair, pair, 0)

    drain_scatter(rowsA, semSA)
    drain_scatter(rowsB, semSB)
    plsc.subcore_barrier()
    base = s * _SLAB
    tail = N_NODES - (NS - 1) * _SLAB

    @pl.when(s < NS - 1)
    def _():
        pltpu.sync_copy(acc.at[pl.ds(base, _SLAB)],
                        part_hbm.at[c, pl.ds(base, _SLAB)])

    @pl.when(s == NS - 1)
    def _():
        pltpu.sync_copy(acc.at[pl.ds((NS - 1) * _SLAB, tail)],
                        part_hbm.at[c, pl.ds((NS - 1) * _SLAB, tail)])


def _tc_norm_body(p_ref, s0_ref, s1_ref, o_ref):
    denom = s0_ref[...] + s1_ref[...] + 1e-12
    o_ref[...] = (p_ref[0] + p_ref[1]) * (1.0 / denom)


def _tc_norm(parts, s0, s1):
    return pl.pallas_call(
        _tc_norm_body,
        out_shape=jax.ShapeDtypeStruct((N_NODES, D), jnp.float32),
    )(parts, s0, s1)


def kernel(h_src, h_dst, edge_index, phi_dt, W_src, W_dst, a):
    a_s = a[:D].reshape(D, 1)
    a_d = a[D:2 * D].reshape(D, 1)
    a_t = a[2 * D:]
    at_blk = jnp.kron(jnp.eye(D, dtype=jnp.float32), a_t.reshape(D_TIME, 1))
    phi2 = phi_dt.reshape(_PHI_ROWS, _PHI_COLS)

    et, es, ed = _tc_attn(h_src, h_dst, W_src, W_dst, a_s, a_d, at_blk, phi2)
    zs = _tc_zs(h_src, W_src)
    et = et.reshape(N_EDGES)
    es = es.reshape(N_NODES)
    ed = ed.reshape(N_NODES)

    ei_flat = edge_index.reshape(2 * N_EDGES)
    exp_e, psums = _sc_scalar(ei_flat, et, es, ed)
    parts = _sc_vector(ei_flat, exp_e, zs)
    s0 = psums[:N_NODES].reshape(N_NODES, 1)
    s1 = psums[N_PAD:N_PAD + N_NODES].reshape(N_NODES, 1)
    return _tc_norm(parts, s0, s1)
